# baseline (speedup 1.0000x reference)
import functools
import math

import jax
import jax.numpy as jnp
from jax import lax
from jax.experimental import pallas as pl
from jax.experimental.pallas import tpu as pltpu
from jax.experimental.pallas import tpu_sc as plsc

VOCAB = 1000000
EMB = 64
SCALE = math.sqrt(EMB)

NC = 2
NS = 16
NW = NC * NS

CHUNK = 128
NBUF = 4
NOB = 2
LANES = 16


def _body(tok_hbm, table_hbm, out_hbm, idx_v, gath_v, outb_v, gsem, osem):
  c = lax.axis_index("c")
  s = lax.axis_index("s")
  wid = s * NC + c
  nch = tok_hbm.shape[1]
  base = wid * nch * CHUNK

  pltpu.sync_copy(tok_hbm.at[wid], idx_v)

  for b in range(NBUF):
    pltpu.async_copy(table_hbm.at[idx_v.at[b]], gath_v.at[b], gsem.at[b])

  def group(g, carry):
    for b in range(NBUF):
      j = g * NBUF + b
      pltpu.make_async_copy(
          table_hbm.at[idx_v.at[j]], gath_v.at[b], gsem.at[b]).wait()

      ob = j % NOB

      @pl.when(j >= NOB)
      def _():
        pltpu.make_async_copy(
            outb_v.at[ob],
            out_hbm.at[pl.ds(base + (j - NOB) * CHUNK, CHUNK)],
            osem.at[ob]).wait()

      def scale_row(i, carry2):
        for e in range(EMB // LANES):
          sl = pl.ds(e * LANES, LANES)
          outb_v[ob, i, sl] = gath_v[b, i, sl]
        return carry2

      lax.fori_loop(0, CHUNK, scale_row, 0, unroll=2)

      pltpu.async_copy(
          outb_v.at[ob],
          out_hbm.at[pl.ds(base + j * CHUNK, CHUNK)],
          osem.at[ob])

      nj = j + NBUF

      @pl.when(nj < nch)
      def _():
        pltpu.async_copy(table_hbm.at[idx_v.at[nj]], gath_v.at[b], gsem.at[b])

    return carry

  lax.fori_loop(0, nch // NBUF, group, 0)

  for k in range(NOB):
    j = nch - NOB + k
    pltpu.make_async_copy(
        outb_v.at[j % NOB],
        out_hbm.at[pl.ds(base + j * CHUNK, CHUNK)],
        osem.at[j % NOB]).wait()


def _widen_body(tt_ref, out_ref):
  eye = jnp.eye(EMB, dtype=jnp.float32) * SCALE
  t = jax.lax.dot_general(tt_ref[...], eye, (((0,), (0,)), ((), ())),
                          preferred_element_type=jnp.float32)
  out_ref[...] = jnp.concatenate([t, t], axis=1)


_VB = 2048


def _widen(table_t):
  grid = (VOCAB + _VB - 1) // _VB
  return pl.pallas_call(
      _widen_body,
      grid=(grid,),
      in_specs=[pl.BlockSpec((EMB, _VB), lambda i: (0, i))],
      out_specs=pl.BlockSpec((_VB, 2 * EMB), lambda i: (i, 0)),
      out_shape=jax.ShapeDtypeStruct((VOCAB, 2 * EMB), jnp.float32),
  )(table_t)


@jax.jit
def kernel(tokens, table):
  n = tokens.shape[0] * tokens.shape[1]
  assert n % (NW * CHUNK) == 0
  nch = n // (NW * CHUNK)
  idx = jnp.reshape(tokens.astype(jnp.int32), (NW, nch, CHUNK))
  table_dbl = _widen(jnp.transpose(table))

  mesh = plsc.VectorSubcoreMesh(
      core_axis_name="c", subcore_axis_name="s", num_cores=NC, num_subcores=NS)
  out = pl.kernel(
      _body,
      out_type=jax.ShapeDtypeStruct((n, EMB), jnp.float32),
      mesh=mesh,
      scratch_types=[
          pltpu.VMEM((nch, CHUNK), jnp.int32),
          pltpu.VMEM((NBUF, CHUNK, 2 * EMB), jnp.float32),
          pltpu.VMEM((NOB, CHUNK, EMB), jnp.float32),
          pltpu.SemaphoreType.DMA((NBUF,)),
          pltpu.SemaphoreType.DMA((NOB,)),
      ],
  )(idx, table_dbl)
  return jnp.reshape(out, (*tokens.shape, EMB))


# speedup vs baseline: 1.5220x; 1.5220x over previous
import functools
import math

import jax
import jax.numpy as jnp
from jax import lax
from jax.experimental import pallas as pl
from jax.experimental.pallas import tpu as pltpu
from jax.experimental.pallas import tpu_sc as plsc

VOCAB = 1000000
EMB = 64
SCALE = math.sqrt(EMB)

NC = 2
NS = 16
NW = NC * NS

CHUNK = 128
NBUF = 4
NOB = 2
LANES = 16


def _body(tok_hbm, table_hbm, out_hbm, idx_v, gath_v, outb_v, gsem, osem):
  c = lax.axis_index("c")
  s = lax.axis_index("s")
  wid = s * NC + c
  nch = tok_hbm.shape[1]
  base = wid * nch * CHUNK

  pltpu.sync_copy(tok_hbm.at[wid], idx_v)

  for b in range(NBUF):
    pltpu.async_copy(table_hbm.at[idx_v.at[b]], gath_v.at[b], gsem.at[b])

  def group(g, carry):
    for b in range(NBUF):
      j = g * NBUF + b
      pltpu.make_async_copy(
          table_hbm.at[idx_v.at[j]], gath_v.at[b], gsem.at[b]).wait()

      ob = j % NOB

      @pl.when(j >= NOB)
      def _():
        pltpu.make_async_copy(
            outb_v.at[ob],
            out_hbm.at[pl.ds(base + (j - NOB) * CHUNK, CHUNK)],
            osem.at[ob]).wait()

      def scale_row(i, carry2):
        for e in range(EMB // LANES):
          sl = pl.ds(e * LANES, LANES)
          outb_v[ob, i, sl] = gath_v[b, i, sl]
        return carry2

      lax.fori_loop(0, CHUNK, scale_row, 0, unroll=2)

      pltpu.async_copy(
          outb_v.at[ob],
          out_hbm.at[pl.ds(base + j * CHUNK, CHUNK)],
          osem.at[ob])

      nj = j + NBUF

      @pl.when(nj < nch)
      def _():
        pltpu.async_copy(table_hbm.at[idx_v.at[nj]], gath_v.at[b], gsem.at[b])

    return carry

  lax.fori_loop(0, nch // NBUF, group, 0)

  for k in range(NOB):
    j = nch - NOB + k
    pltpu.make_async_copy(
        outb_v.at[j % NOB],
        out_hbm.at[pl.ds(base + j * CHUNK, CHUNK)],
        osem.at[j % NOB]).wait()


def _widen_body(tt_ref, out_ref):
  eye = jnp.eye(EMB, dtype=jnp.float32) * SCALE
  t = jax.lax.dot_general(tt_ref[...], eye, (((0,), (0,)), ((), ())),
                          preferred_element_type=jnp.float32)
  out_ref[...] = jnp.concatenate([t, t], axis=1)


_VB = 8192


def _widen(table_t):
  grid = (VOCAB + _VB - 1) // _VB
  return pl.pallas_call(
      _widen_body,
      grid=(grid,),
      in_specs=[pl.BlockSpec((EMB, _VB), lambda i: (0, i))],
      out_specs=pl.BlockSpec((_VB, 2 * EMB), lambda i: (i, 0)),
      out_shape=jax.ShapeDtypeStruct((VOCAB, 2 * EMB), jnp.float32),
  )(table_t)


@jax.jit
def kernel(tokens, table):
  n = tokens.shape[0] * tokens.shape[1]
  assert n % (NW * CHUNK) == 0
  nch = n // (NW * CHUNK)
  idx = jnp.reshape(tokens.astype(jnp.int32), (NW, nch, CHUNK))
  table_dbl = _widen(jnp.transpose(table))

  mesh = plsc.VectorSubcoreMesh(
      core_axis_name="c", subcore_axis_name="s", num_cores=NC, num_subcores=NS)
  out = pl.kernel(
      _body,
      out_type=jax.ShapeDtypeStruct((n, EMB), jnp.float32),
      mesh=mesh,
      scratch_types=[
          pltpu.VMEM((nch, CHUNK), jnp.int32),
          pltpu.VMEM((NBUF, CHUNK, 2 * EMB), jnp.float32),
          pltpu.VMEM((NOB, CHUNK, EMB), jnp.float32),
          pltpu.SemaphoreType.DMA((NBUF,)),
          pltpu.SemaphoreType.DMA((NOB,)),
      ],
  )(idx, table_dbl)
  return jnp.reshape(out, (*tokens.shape, EMB))
